# baseline (speedup 1.0000x reference)
import functools

import jax
import jax.numpy as jnp
from jax import lax
from jax.experimental import pallas as pl
from jax.experimental.pallas import tpu as pltpu
from jax.experimental.pallas import tpu_sc as plsc

N_GAME = 50000
N_STATE = 10000
D_IN = 5
H = 512
OUT = 8

NC = 2
NS = 16
NW = NC * NS
CHUNK = 128

NG_ACC = 50176
NS_ACC = 10240
DG = 16


def _pad_rows(x, n):
    return jnp.pad(x, ((0, n - x.shape[0]),) + ((0, 0),) * (x.ndim - 1))


def _make_degrees(e_pads, n_accs):
    mesh = plsc.VectorSubcoreMesh(
        core_axis_name="c", subcore_axis_name="s", num_cores=NC, num_subcores=NS)
    out_type = tuple(jax.ShapeDtypeStruct((NC * n,), jnp.float32) for n in n_accs)
    scratch = [
        pltpu.VMEM((CHUNK,), jnp.int32),
        pltpu.VMEM((CHUNK,), jnp.float32),
        pltpu.VMEM((64,), jnp.float32),
    ] + [pltpu.VMEM_SHARED((n,), jnp.float32) for n in n_accs]

    def body(col0, col1, col2, ones_hbm, zeros_hbm, out0, out1, out2,
             cidx, obuf, sbuf, acc0, acc1, acc2):
        cid = lax.axis_index("c")
        sid = lax.axis_index("s")
        w = cid * NS + sid
        cols = (col0, col1, col2)
        outs = (out0, out1, out2)
        pltpu.sync_copy(ones_hbm, obuf)
        pltpu.sync_copy(zeros_hbm, sbuf)
        acc_refs = (acc0, acc1, acc2)
        if True:
            for i in range(3):
                rows_w = n_accs[i] // NS

                def zbody(t, _, i=i, rows_w=rows_w):
                    pltpu.sync_copy(
                        sbuf, acc_refs[i].at[pl.ds(sid * rows_w + t * 64, 64)])
                    return 0

                lax.fori_loop(0, rows_w // 64, zbody, 0)
            plsc.subcore_barrier()
            for i in range(3):
                ew = e_pads[i] // NW
                nk = ew // CHUNK

                def ebody(k, _, i=i):
                    base = w * ew + k * CHUNK
                    pltpu.sync_copy(cols[i].at[pl.ds(base, CHUNK)], cidx)
                    pltpu.sync_copy(obuf, acc_refs[i].at[cidx], add=True)
                    return 0

                lax.fori_loop(0, nk, ebody, 0)
            plsc.subcore_barrier()
            for i in range(3):
                rows_w = n_accs[i] // NS

                def cbody(t, _, i=i, rows_w=rows_w):
                    pltpu.sync_copy(
                        acc_refs[i].at[pl.ds(sid * rows_w + t * 64, 64)], sbuf)
                    pltpu.sync_copy(
                        sbuf,
                        outs[i].at[pl.ds(
                            cid * n_accs[i] + sid * rows_w + t * 64, 64)])
                    return 0

                lax.fori_loop(0, rows_w // 64, cbody, 0)

    return pl.kernel(body, out_type=out_type, mesh=mesh, scratch_types=scratch)


def _make_apply(n_chunks, dc, e_pad, n_acc, weighted):
    mesh = plsc.VectorSubcoreMesh(
        core_axis_name="c", subcore_axis_name="s", num_cores=NC, num_subcores=NS)
    out_type = jax.ShapeDtypeStruct((NC, n_chunks, n_acc, dc), jnp.float32)
    rows_w = n_acc // NS
    ew = e_pad // NW
    nk = ew // CHUNK
    scratch = [
        pltpu.VMEM((CHUNK,), jnp.int32),
        pltpu.VMEM((CHUNK,), jnp.int32),
        pltpu.VMEM((CHUNK,), jnp.int32),
        pltpu.VMEM((CHUNK,), jnp.int32),
        pltpu.VMEM((CHUNK, dc), jnp.float32),
        pltpu.VMEM((CHUNK, dc), jnp.float32),
        pltpu.VMEM((64, dc), jnp.float32),
        pltpu.SemaphoreType.DMA,
        pltpu.SemaphoreType.DMA,
        pltpu.SemaphoreType.DMA,
        pltpu.SemaphoreType.DMA,
    ]
    if weighted:
        scratch.append(pltpu.VMEM((CHUNK,), jnp.float32))
        scratch.append(pltpu.VMEM((CHUNK,), jnp.float32))
    scratch.append(pltpu.VMEM_SHARED((n_acc, dc), jnp.float32))
    cparams = pltpu.CompilerParams(use_tc_tiling_on_sc=(dc % 128 == 0),
                                   needs_layout_passes=False)

    def body(*refs):
        xs = refs[:n_chunks]
        row_hbm, col_hbm = refs[n_chunks], refs[n_chunks + 1]
        p = n_chunks + 2
        if weighted:
            wts_hbm = refs[p]
            p += 1
        zeros_hbm = refs[p]
        out = refs[p + 1]
        (ridx0, cidx0, ridx1, cidx1, dbuf0, dbuf1, zbuf,
         gsem0, gsem1, ssem0, ssem1) = refs[p + 2:p + 13]
        if weighted:
            wbuf0, wbuf1 = refs[p + 13], refs[p + 14]
            wbufs = (wbuf0, wbuf1)
        acc = refs[-1]
        cid = lax.axis_index("c")
        sid = lax.axis_index("s")
        w = cid * NS + sid
        ridxs, cidxs = (ridx0, ridx1), (cidx0, cidx1)
        dbufs = (dbuf0, dbuf1)
        gsems, ssems = (gsem0, gsem1), (ssem0, ssem1)

        pltpu.sync_copy(zeros_hbm, zbuf)

        def load_idx(b, k):
            base = w * ew + k * CHUNK
            pltpu.sync_copy(row_hbm.at[pl.ds(base, CHUNK)], ridxs[b])
            pltpu.sync_copy(col_hbm.at[pl.ds(base, CHUNK)], cidxs[b])
            if weighted:
                pltpu.sync_copy(wts_hbm.at[pl.ds(base, CHUNK)], wbufs[b])

        def mul(b):
            def mbody(e, _):
                ws = plsc.load_gather(
                    wbufs[b], [jnp.full((16,), e, jnp.int32)])
                for q in range(dc // 16):
                    dbufs[b][e, pl.ds(q * 16, 16)] = (
                        dbufs[b][e, pl.ds(q * 16, 16)] * ws)
                return 0

            lax.fori_loop(0, CHUNK, mbody, 0)

        for c in range(n_chunks):
            def zbody(t, _):
                pltpu.sync_copy(
                    zbuf, acc.at[pl.ds(sid * rows_w + t * 64, 64)])
                return 0

            lax.fori_loop(0, rows_w // 64, zbody, 0)
            plsc.subcore_barrier()

            load_idx(0, 0)
            pltpu.async_copy(xs[c].at[ridx0], dbuf0, gsem0)

            def ebody(t, _, c=c):
                k = 2 * t
                load_idx(1, k + 1)
                pltpu.async_copy(xs[c].at[ridx1], dbuf1, gsem1)
                pltpu.make_async_copy(xs[c].at[ridx0], dbuf0, gsem0).wait()
                if weighted:
                    mul(0)
                pltpu.sync_copy(dbuf0, acc.at[cidx0], add=True)
                load_idx(0, jnp.minimum(k + 2, nk - 1))
                pltpu.async_copy(xs[c].at[ridx0], dbuf0, gsem0)
                pltpu.make_async_copy(xs[c].at[ridx1], dbuf1, gsem1).wait()
                if weighted:
                    mul(1)
                pltpu.sync_copy(dbuf1, acc.at[cidx1], add=True)
                return 0

            lax.fori_loop(0, nk // 2, ebody, 0)
            pltpu.make_async_copy(xs[c].at[ridx0], dbuf0, gsem0).wait()
            plsc.subcore_barrier()

            def cbody(t, _, c=c):
                r0 = sid * rows_w + t * 64
                pltpu.sync_copy(acc.at[pl.ds(r0, 64)],
                                dbuf0.at[pl.ds(0, 64)])
                pltpu.sync_copy(dbuf0.at[pl.ds(0, 64)],
                                out.at[cid, c, pl.ds(r0, 64)])
                return 0

            lax.fori_loop(0, rows_w // 64, cbody, 0)
            plsc.subcore_barrier()

    return pl.kernel(body, out_type=out_type, mesh=mesh, scratch_types=scratch,
                     compiler_params=cparams)


_R = 512


def _full(shape):
    return pl.BlockSpec(shape, lambda i: (0,) * len(shape))


def _dot(a, b):
    return jnp.dot(a, b, preferred_element_type=jnp.float32)


def _tg_body(x_ref, p1_ref, p2_ref, dinv_ref, w_ref, b_ref, out_ref):
    dinv = dinv_ref[...]
    h1 = dinv * (p1_ref[0] + p1_ref[1])
    h2 = dinv * (p2_ref[0] + p2_ref[1])
    a = jnp.concatenate([x_ref[...], h1, h2], axis=1)
    g = jnp.maximum(_dot(a, w_ref[...]) + b_ref[...], 0.0)
    for c in range(4):
        out_ref[c] = g[:, c * 128:(c + 1) * 128]


def _tc_game(x16, p1, p2, dinv, w48, bias):
    grid = (NG_ACC // _R,)
    return pl.pallas_call(
        _tg_body,
        grid=grid,
        in_specs=[
            pl.BlockSpec((_R, DG), lambda i: (i, 0)),
            pl.BlockSpec((2, _R, DG), lambda i: (0, i, 0)),
            pl.BlockSpec((2, _R, DG), lambda i: (0, i, 0)),
            pl.BlockSpec((_R, 1), lambda i: (i, 0)),
            _full((3 * DG, H)),
            _full((1, H)),
        ],
        out_specs=pl.BlockSpec((4, _R, 128), lambda i: (0, i, 0)),
        out_shape=jax.ShapeDtypeStruct((4, NG_ACC, 128), jnp.float32),
    )(x16, p1, p2, dinv, w48, bias)


def _combine(p_ref, c):
    return jnp.concatenate([p_ref[0, c2] + p_ref[1, c2] for c2 in range(4)],
                           axis=1) if c is None else p_ref[0, c] + p_ref[1, c]


def _ts_body(a3_ref, s4_ref, x8_ref, dinv_ref, icnt_ref, w3rel_ref, w3root_ref,
             w4l_ref, w4r_ref, b3_ref, b4_ref, s4_out, s4s_out):
    agg = _combine(a3_ref, None)
    s3 = jnp.maximum(
        _dot(agg, w3rel_ref[...]) + _dot(x8_ref[...], w3root_ref[...])
        + b3_ref[...], 0.0)
    mean = _combine(s4_ref, None) * icnt_ref[...]
    s4 = jnp.maximum(
        _dot(mean, w4l_ref[...]) + _dot(s3, w4r_ref[...]) + b4_ref[...], 0.0)
    s4_out[...] = s4
    dinv = dinv_ref[...]
    for c in range(4):
        s4s_out[c] = dinv * s4[:, c * 128:(c + 1) * 128]


def _tc_s34(a3p, s4p, x8, dinv_s, inv_cnt, w3rel, w3root8, w4l, w4r, b3, b4):
    grid = (NS_ACC // _R,)
    return pl.pallas_call(
        _ts_body,
        grid=grid,
        in_specs=[
            pl.BlockSpec((2, 4, _R, 128), lambda i: (0, 0, i, 0)),
            pl.BlockSpec((2, 4, _R, 128), lambda i: (0, 0, i, 0)),
            pl.BlockSpec((_R, 8), lambda i: (i, 0)),
            pl.BlockSpec((_R, 1), lambda i: (i, 0)),
            pl.BlockSpec((_R, 1), lambda i: (i, 0)),
            _full((H, H)),
            _full((8, H)),
            _full((H, H)),
            _full((H, H)),
            _full((1, H)),
            _full((1, H)),
        ],
        out_specs=[
            pl.BlockSpec((_R, H), lambda i: (i, 0)),
            pl.BlockSpec((4, _R, 128), lambda i: (0, i, 0)),
        ],
        out_shape=[
            jax.ShapeDtypeStruct((NS_ACC, H), jnp.float32),
            jax.ShapeDtypeStruct((4, NS_ACC, 128), jnp.float32),
        ],
    )(a3p, s4p, x8, dinv_s, inv_cnt, w3rel, w3root8, w4l, w4r, b3, b4)


def _thc_body(qp_ref, dinv_ref, h_out, hs_out):
    dinv = dinv_ref[...]
    for c in range(4):
        hc = dinv * _combine(qp_ref, c)
        h_out[:, c * 128:(c + 1) * 128] = hc
        hs_out[c] = dinv * hc


def _tc_hop(qp, dinv_s):
    grid = (NS_ACC // _R,)
    return pl.pallas_call(
        _thc_body,
        grid=grid,
        in_specs=[
            pl.BlockSpec((2, 4, _R, 128), lambda i: (0, 0, i, 0)),
            pl.BlockSpec((_R, 1), lambda i: (i, 0)),
        ],
        out_specs=[
            pl.BlockSpec((_R, H), lambda i: (i, 0)),
            pl.BlockSpec((4, _R, 128), lambda i: (0, i, 0)),
        ],
        out_shape=[
            jax.ShapeDtypeStruct((NS_ACC, H), jnp.float32),
            jax.ShapeDtypeStruct((4, NS_ACC, 128), jnp.float32),
        ],
    )(qp, dinv_s)


def _tf_body(s4_ref, h1_ref, h2_ref, q3_ref, dinv_ref, w0_ref, w1_ref, w2_ref,
             w3_ref, b_ref, wlin_ref, blin_ref, out_ref):
    h3 = dinv_ref[...] * _combine(q3_ref, None)
    hh = (_dot(s4_ref[...], w0_ref[...]) + _dot(h1_ref[...], w1_ref[...])
          + _dot(h2_ref[...], w2_ref[...]) + _dot(h3, w3_ref[...])
          + b_ref[...])
    hh = jnp.maximum(hh, 0.0)
    out_ref[...] = _dot(hh, wlin_ref[...]) + blin_ref[...]


def _tc_final(s4, h1, h2, q3p, dinv_s, w0, w1, w2, w3, bsum, wlin, blin):
    grid = (NS_ACC // _R,)
    return pl.pallas_call(
        _tf_body,
        grid=grid,
        in_specs=[
            pl.BlockSpec((_R, H), lambda i: (i, 0)),
            pl.BlockSpec((_R, H), lambda i: (i, 0)),
            pl.BlockSpec((_R, H), lambda i: (i, 0)),
            pl.BlockSpec((2, 4, _R, 128), lambda i: (0, 0, i, 0)),
            pl.BlockSpec((_R, 1), lambda i: (i, 0)),
            _full((H, H)),
            _full((H, H)),
            _full((H, H)),
            _full((H, H)),
            _full((1, H)),
            _full((H, OUT)),
            _full((1, OUT)),
        ],
        out_specs=pl.BlockSpec((_R, OUT), lambda i: (i, 0)),
        out_shape=jax.ShapeDtypeStruct((NS_ACC, OUT), jnp.float32),
    )(s4, h1, h2, q3p, dinv_s, w0, w1, w2, w3, bsum, wlin, blin)


def _pad_edges(idx_arr, e_pad, fill):
    return jnp.pad(idx_arr, (0, e_pad - idx_arr.shape[0]),
                   constant_values=fill)


def _epad(e):
    g = 2 * NW * CHUNK
    return ((e + g - 1) // g) * g


def _safe_rsqrt(deg):
    return jnp.where(deg > 0, lax.rsqrt(jnp.maximum(deg, 1.0)), 0.0)


def kernel(game_x, state_x, edge_index_v_v, edge_index_history_v_s,
           edge_attr_history_v_s, edge_index_in_v_s, edge_index_s_s,
           W1_0, b1_0, W1_1, b1_1, W1_2, b1_2, W2_0, b2_0, W2_1, b2_1,
           W2_2, b2_2, W2_3, b2_3, W3_rel, b3_rel, W3_root, W4_l, b4_l,
           W4_r, Wlin, blin):
    f32 = jnp.float32
    e_vv = _epad(edge_index_v_v.shape[1])
    e_hist = _epad(edge_index_history_v_s.shape[1])
    e_in = _epad(edge_index_in_v_s.shape[1])
    e_ss = _epad(edge_index_s_s.shape[1])

    row_vv = _pad_edges(edge_index_v_v[0], e_vv, 0)
    col_vv = _pad_edges(edge_index_v_v[1], e_vv, N_GAME)
    row_hist = _pad_edges(edge_index_history_v_s[0], e_hist, 0)
    col_hist = _pad_edges(edge_index_history_v_s[1], e_hist, N_STATE)
    w_hist = jnp.pad(edge_attr_history_v_s, (0, e_hist - edge_attr_history_v_s.shape[0]))
    row_in = _pad_edges(edge_index_in_v_s[0], e_in, 0)
    col_in = _pad_edges(edge_index_in_v_s[1], e_in, N_STATE)
    row_ss = _pad_edges(edge_index_s_s[0], e_ss, 0)
    col_ss = _pad_edges(edge_index_s_s[1], e_ss, N_STATE)

    ones128 = jnp.ones((CHUNK,), f32)
    zer_deg = jnp.zeros((64,), f32)

    deg_k = _make_degrees((e_vv, e_ss, e_in), (NG_ACC, NS_ACC, NS_ACC))
    degp_vv, degp_ss, cntp_in = deg_k(col_vv, col_ss, col_in, ones128, zer_deg)
    deg_vv = degp_vv[:NG_ACC] + degp_vv[NG_ACC:]
    deg_ss = degp_ss[:NS_ACC] + degp_ss[NS_ACC:]
    cnt_in = cntp_in[:NS_ACC] + cntp_in[NS_ACC:]
    dinv_g = _safe_rsqrt(deg_vv)[:, None]
    dinv_s = _safe_rsqrt(deg_ss)[:, None]
    inv_cnt = (1.0 / jnp.maximum(cnt_in, 1.0))[:, None]

    x16 = _pad_rows(jnp.pad(game_x, ((0, 0), (0, DG - D_IN))), NG_ACC)
    x_s = x16 * dinv_g
    zer_g = jnp.zeros((64, DG), f32)
    apply_g = _make_apply(1, DG, e_vv, NG_ACC, False)
    p1 = apply_g(x_s, row_vv, col_vv, zer_g)[:, 0]
    h1_s = (dinv_g * dinv_g) * (p1[0] + p1[1])
    p2 = apply_g(h1_s, row_vv, col_vv, zer_g)[:, 0]
    w48 = jnp.concatenate([
        jnp.pad(W1_0, ((0, DG - D_IN), (0, 0))),
        jnp.pad(W1_1, ((0, DG - D_IN), (0, 0))),
        jnp.pad(W1_2, ((0, DG - D_IN), (0, 0)))], axis=0)
    b1 = (b1_0 + b1_1 + b1_2)[None, :]
    g4 = _tc_game(x16, p1, p2, dinv_g, w48, b1)
    gx = [g4[c] for c in range(4)]

    zer_s = jnp.zeros((64, 128), f32)
    apply_h = _make_apply(4, 128, e_hist, NS_ACC, True)
    a3p = apply_h(*gx, row_hist, col_hist, w_hist, zer_s)
    apply_i = _make_apply(4, 128, e_in, NS_ACC, False)
    s4p = apply_i(*gx, row_in, col_in, zer_s)

    x8 = _pad_rows(jnp.pad(state_x, ((0, 0), (0, 8 - D_IN))), NS_ACC)
    w3root8 = jnp.pad(W3_root, ((0, 8 - D_IN), (0, 0)))
    s4, s4s = _tc_s34(a3p, s4p, x8, dinv_s, inv_cnt, W3_rel, w3root8,
                      W4_l, W4_r, b3_rel[None, :], b4_l[None, :])

    apply_s = _make_apply(4, 128, e_ss, NS_ACC, False)
    q1 = apply_s(s4s[0], s4s[1], s4s[2], s4s[3], row_ss, col_ss, zer_s)
    h1, h1s = _tc_hop(q1, dinv_s)
    q2 = apply_s(h1s[0], h1s[1], h1s[2], h1s[3], row_ss, col_ss, zer_s)
    h2, h2s = _tc_hop(q2, dinv_s)
    q3 = apply_s(h2s[0], h2s[1], h2s[2], h2s[3], row_ss, col_ss, zer_s)

    bsum = (b2_0 + b2_1 + b2_2 + b2_3)[None, :]
    out = _tc_final(s4, h1, h2, q3, dinv_s, W2_0, W2_1, W2_2, W2_3,
                    bsum, Wlin, blin[None, :])
    return out[:N_STATE]


# speedup vs baseline: 1.5241x; 1.5241x over previous
import functools

import jax
import jax.numpy as jnp
from jax import lax
from jax.experimental import pallas as pl
from jax.experimental.pallas import tpu as pltpu
from jax.experimental.pallas import tpu_sc as plsc

N_GAME = 50000
N_STATE = 10000
D_IN = 5
H = 512
OUT = 8

NC = 2
NS = 16
NW = NC * NS
CHUNK = 128

NG_ACC = 50176
NS_ACC = 10240
DG = 16


def _pad_rows(x, n):
    return jnp.pad(x, ((0, n - x.shape[0]),) + ((0, 0),) * (x.ndim - 1))


def _make_degrees(e_pads, n_accs):
    mesh = plsc.VectorSubcoreMesh(
        core_axis_name="c", subcore_axis_name="s", num_cores=NC, num_subcores=NS)
    out_type = tuple(jax.ShapeDtypeStruct((NC * n,), jnp.float32) for n in n_accs)
    scratch = [
        pltpu.VMEM((CHUNK,), jnp.int32),
        pltpu.VMEM((CHUNK,), jnp.float32),
        pltpu.VMEM((64,), jnp.float32),
    ] + [pltpu.VMEM_SHARED((n,), jnp.float32) for n in n_accs]

    def body(col0, col1, col2, ones_hbm, zeros_hbm, out0, out1, out2,
             cidx, obuf, sbuf, acc0, acc1, acc2):
        cid = lax.axis_index("c")
        sid = lax.axis_index("s")
        w = cid * NS + sid
        cols = (col0, col1, col2)
        outs = (out0, out1, out2)
        pltpu.sync_copy(ones_hbm, obuf)
        pltpu.sync_copy(zeros_hbm, sbuf)
        acc_refs = (acc0, acc1, acc2)
        if True:
            for i in range(3):
                rows_w = n_accs[i] // NS

                def zbody(t, _, i=i, rows_w=rows_w):
                    pltpu.sync_copy(
                        sbuf, acc_refs[i].at[pl.ds(sid * rows_w + t * 64, 64)])
                    return 0

                lax.fori_loop(0, rows_w // 64, zbody, 0)
            plsc.subcore_barrier()
            for i in range(3):
                ew = e_pads[i] // NW
                nk = ew // CHUNK

                def ebody(k, _, i=i):
                    base = w * ew + k * CHUNK
                    pltpu.sync_copy(cols[i].at[pl.ds(base, CHUNK)], cidx)
                    pltpu.sync_copy(obuf, acc_refs[i].at[cidx], add=True)
                    return 0

                lax.fori_loop(0, nk, ebody, 0)
            plsc.subcore_barrier()
            for i in range(3):
                rows_w = n_accs[i] // NS

                def cbody(t, _, i=i, rows_w=rows_w):
                    pltpu.sync_copy(
                        acc_refs[i].at[pl.ds(sid * rows_w + t * 64, 64)], sbuf)
                    pltpu.sync_copy(
                        sbuf,
                        outs[i].at[pl.ds(
                            cid * n_accs[i] + sid * rows_w + t * 64, 64)])
                    return 0

                lax.fori_loop(0, rows_w // 64, cbody, 0)

    return pl.kernel(body, out_type=out_type, mesh=mesh, scratch_types=scratch)


def _make_apply(n_chunks, dc, e_pad, n_acc, weighted):
    mesh = plsc.VectorSubcoreMesh(
        core_axis_name="c", subcore_axis_name="s", num_cores=NC, num_subcores=NS)
    out_type = jax.ShapeDtypeStruct((NC, n_chunks, n_acc, dc), jnp.float32)
    rows_w = n_acc // NS
    ew = e_pad // NW
    nk = ew // CHUNK
    scratch = [
        pltpu.VMEM((CHUNK,), jnp.int32),
        pltpu.VMEM((CHUNK,), jnp.int32),
        pltpu.VMEM((CHUNK, dc), jnp.float32),
        pltpu.VMEM((64, dc), jnp.float32),
    ]
    if weighted:
        scratch.append(pltpu.VMEM((CHUNK,), jnp.float32))
    scratch.append(pltpu.VMEM_SHARED((n_acc, dc), jnp.float32))
    cparams = pltpu.CompilerParams(use_tc_tiling_on_sc=(dc % 128 == 0),
                                   needs_layout_passes=False)

    def body(*refs):
        xs = refs[:n_chunks]
        row_hbm, col_hbm = refs[n_chunks], refs[n_chunks + 1]
        p = n_chunks + 2
        if weighted:
            wts_hbm = refs[p]
            p += 1
        zeros_hbm = refs[p]
        out = refs[p + 1]
        ridx, cidx, dbuf, zbuf = (refs[p + 2], refs[p + 3], refs[p + 4],
                                  refs[p + 5])
        if weighted:
            wbuf = refs[p + 6]
        acc = refs[-1]
        cid = lax.axis_index("c")
        sid = lax.axis_index("s")
        w = cid * NS + sid

        if True:
            pltpu.sync_copy(zeros_hbm, zbuf)
            for c in range(n_chunks):
                def zbody(t, _):
                    pltpu.sync_copy(
                        zbuf, acc.at[pl.ds(sid * rows_w + t * 64, 64)])
                    return 0

                lax.fori_loop(0, rows_w // 64, zbody, 0)
                plsc.subcore_barrier()

                def ebody(k, _, c=c):
                    base = w * ew + k * CHUNK
                    pltpu.sync_copy(row_hbm.at[pl.ds(base, CHUNK)], ridx)
                    pltpu.sync_copy(col_hbm.at[pl.ds(base, CHUNK)], cidx)
                    pltpu.sync_copy(xs[c].at[ridx], dbuf)
                    if weighted:
                        pltpu.sync_copy(wts_hbm.at[pl.ds(base, CHUNK)], wbuf)

                        def mbody(e, _):
                            ws = plsc.load_gather(
                                wbuf, [jnp.full((16,), e, jnp.int32)])
                            for j in range(dc // 16):
                                dbuf[e, pl.ds(j * 16, 16)] = (
                                    dbuf[e, pl.ds(j * 16, 16)] * ws)
                            return 0

                        lax.fori_loop(0, CHUNK, mbody, 0)
                    pltpu.sync_copy(dbuf, acc.at[cidx], add=True)
                    return 0

                lax.fori_loop(0, nk, ebody, 0)
                plsc.subcore_barrier()

                def cbody(t, _, c=c):
                    r0 = sid * rows_w + t * 64
                    pltpu.sync_copy(acc.at[pl.ds(r0, 64)],
                                    dbuf.at[pl.ds(0, 64)])
                    pltpu.sync_copy(dbuf.at[pl.ds(0, 64)],
                                    out.at[cid, c, pl.ds(r0, 64)])
                    return 0

                lax.fori_loop(0, rows_w // 64, cbody, 0)
                plsc.subcore_barrier()

    return pl.kernel(body, out_type=out_type, mesh=mesh, scratch_types=scratch,
                     compiler_params=cparams)


_R = 512


def _full(shape):
    return pl.BlockSpec(shape, lambda i: (0,) * len(shape))


def _dot(a, b):
    return jnp.dot(a, b, preferred_element_type=jnp.float32)


def _tg_body(x_ref, p1_ref, p2_ref, dinv_ref, w_ref, b_ref, out_ref):
    dinv = dinv_ref[...]
    h1 = dinv * (p1_ref[0] + p1_ref[1])
    h2 = dinv * (p2_ref[0] + p2_ref[1])
    a = jnp.concatenate([x_ref[...], h1, h2], axis=1)
    g = jnp.maximum(_dot(a, w_ref[...]) + b_ref[...], 0.0)
    for c in range(4):
        out_ref[c] = g[:, c * 128:(c + 1) * 128]


def _tc_game(x16, p1, p2, dinv, w48, bias):
    grid = (NG_ACC // _R,)
    return pl.pallas_call(
        _tg_body,
        grid=grid,
        in_specs=[
            pl.BlockSpec((_R, DG), lambda i: (i, 0)),
            pl.BlockSpec((2, _R, DG), lambda i: (0, i, 0)),
            pl.BlockSpec((2, _R, DG), lambda i: (0, i, 0)),
            pl.BlockSpec((_R, 1), lambda i: (i, 0)),
            _full((3 * DG, H)),
            _full((1, H)),
        ],
        out_specs=pl.BlockSpec((4, _R, 128), lambda i: (0, i, 0)),
        out_shape=jax.ShapeDtypeStruct((4, NG_ACC, 128), jnp.float32),
    )(x16, p1, p2, dinv, w48, bias)


def _combine(p_ref, c):
    return jnp.concatenate([p_ref[0, c2] + p_ref[1, c2] for c2 in range(4)],
                           axis=1) if c is None else p_ref[0, c] + p_ref[1, c]


def _ts_body(a3_ref, s4_ref, x8_ref, dinv_ref, icnt_ref, w3rel_ref, w3root_ref,
             w4l_ref, w4r_ref, b3_ref, b4_ref, s4_out, s4s_out):
    agg = _combine(a3_ref, None)
    s3 = jnp.maximum(
        _dot(agg, w3rel_ref[...]) + _dot(x8_ref[...], w3root_ref[...])
        + b3_ref[...], 0.0)
    mean = _combine(s4_ref, None) * icnt_ref[...]
    s4 = jnp.maximum(
        _dot(mean, w4l_ref[...]) + _dot(s3, w4r_ref[...]) + b4_ref[...], 0.0)
    s4_out[...] = s4
    dinv = dinv_ref[...]
    for c in range(4):
        s4s_out[c] = dinv * s4[:, c * 128:(c + 1) * 128]


def _tc_s34(a3p, s4p, x8, dinv_s, inv_cnt, w3rel, w3root8, w4l, w4r, b3, b4):
    grid = (NS_ACC // _R,)
    return pl.pallas_call(
        _ts_body,
        grid=grid,
        in_specs=[
            pl.BlockSpec((2, 4, _R, 128), lambda i: (0, 0, i, 0)),
            pl.BlockSpec((2, 4, _R, 128), lambda i: (0, 0, i, 0)),
            pl.BlockSpec((_R, 8), lambda i: (i, 0)),
            pl.BlockSpec((_R, 1), lambda i: (i, 0)),
            pl.BlockSpec((_R, 1), lambda i: (i, 0)),
            _full((H, H)),
            _full((8, H)),
            _full((H, H)),
            _full((H, H)),
            _full((1, H)),
            _full((1, H)),
        ],
        out_specs=[
            pl.BlockSpec((_R, H), lambda i: (i, 0)),
            pl.BlockSpec((4, _R, 128), lambda i: (0, i, 0)),
        ],
        out_shape=[
            jax.ShapeDtypeStruct((NS_ACC, H), jnp.float32),
            jax.ShapeDtypeStruct((4, NS_ACC, 128), jnp.float32),
        ],
    )(a3p, s4p, x8, dinv_s, inv_cnt, w3rel, w3root8, w4l, w4r, b3, b4)


def _thc_body(qp_ref, dinv_ref, h_out, hs_out):
    dinv = dinv_ref[...]
    for c in range(4):
        hc = dinv * _combine(qp_ref, c)
        h_out[:, c * 128:(c + 1) * 128] = hc
        hs_out[c] = dinv * hc


def _tc_hop(qp, dinv_s):
    grid = (NS_ACC // _R,)
    return pl.pallas_call(
        _thc_body,
        grid=grid,
        in_specs=[
            pl.BlockSpec((2, 4, _R, 128), lambda i: (0, 0, i, 0)),
            pl.BlockSpec((_R, 1), lambda i: (i, 0)),
        ],
        out_specs=[
            pl.BlockSpec((_R, H), lambda i: (i, 0)),
            pl.BlockSpec((4, _R, 128), lambda i: (0, i, 0)),
        ],
        out_shape=[
            jax.ShapeDtypeStruct((NS_ACC, H), jnp.float32),
            jax.ShapeDtypeStruct((4, NS_ACC, 128), jnp.float32),
        ],
    )(qp, dinv_s)


def _tf_body(s4_ref, h1_ref, h2_ref, q3_ref, dinv_ref, w0_ref, w1_ref, w2_ref,
             w3_ref, b_ref, wlin_ref, blin_ref, out_ref):
    h3 = dinv_ref[...] * _combine(q3_ref, None)
    hh = (_dot(s4_ref[...], w0_ref[...]) + _dot(h1_ref[...], w1_ref[...])
          + _dot(h2_ref[...], w2_ref[...]) + _dot(h3, w3_ref[...])
          + b_ref[...])
    hh = jnp.maximum(hh, 0.0)
    out_ref[...] = _dot(hh, wlin_ref[...]) + blin_ref[...]


def _tc_final(s4, h1, h2, q3p, dinv_s, w0, w1, w2, w3, bsum, wlin, blin):
    grid = (NS_ACC // _R,)
    return pl.pallas_call(
        _tf_body,
        grid=grid,
        in_specs=[
            pl.BlockSpec((_R, H), lambda i: (i, 0)),
            pl.BlockSpec((_R, H), lambda i: (i, 0)),
            pl.BlockSpec((_R, H), lambda i: (i, 0)),
            pl.BlockSpec((2, 4, _R, 128), lambda i: (0, 0, i, 0)),
            pl.BlockSpec((_R, 1), lambda i: (i, 0)),
            _full((H, H)),
            _full((H, H)),
            _full((H, H)),
            _full((H, H)),
            _full((1, H)),
            _full((H, OUT)),
            _full((1, OUT)),
        ],
        out_specs=pl.BlockSpec((_R, OUT), lambda i: (i, 0)),
        out_shape=jax.ShapeDtypeStruct((NS_ACC, OUT), jnp.float32),
    )(s4, h1, h2, q3p, dinv_s, w0, w1, w2, w3, bsum, wlin, blin)


def _pad_edges(idx_arr, e_pad, fill):
    return jnp.pad(idx_arr, (0, e_pad - idx_arr.shape[0]),
                   constant_values=fill)


def _epad(e):
    g = NW * CHUNK
    return ((e + g - 1) // g) * g


def _safe_rsqrt(deg):
    return jnp.where(deg > 0, lax.rsqrt(jnp.maximum(deg, 1.0)), 0.0)


def kernel(game_x, state_x, edge_index_v_v, edge_index_history_v_s,
           edge_attr_history_v_s, edge_index_in_v_s, edge_index_s_s,
           W1_0, b1_0, W1_1, b1_1, W1_2, b1_2, W2_0, b2_0, W2_1, b2_1,
           W2_2, b2_2, W2_3, b2_3, W3_rel, b3_rel, W3_root, W4_l, b4_l,
           W4_r, Wlin, blin):
    f32 = jnp.float32
    e_vv = _epad(edge_index_v_v.shape[1])
    e_hist = _epad(edge_index_history_v_s.shape[1])
    e_in = _epad(edge_index_in_v_s.shape[1])
    e_ss = _epad(edge_index_s_s.shape[1])

    row_vv = _pad_edges(edge_index_v_v[0], e_vv, 0)
    col_vv = _pad_edges(edge_index_v_v[1], e_vv, N_GAME)
    row_hist = _pad_edges(edge_index_history_v_s[0], e_hist, 0)
    col_hist = _pad_edges(edge_index_history_v_s[1], e_hist, N_STATE)
    w_hist = jnp.pad(edge_attr_history_v_s, (0, e_hist - edge_attr_history_v_s.shape[0]))
    row_in = _pad_edges(edge_index_in_v_s[0], e_in, 0)
    col_in = _pad_edges(edge_index_in_v_s[1], e_in, N_STATE)
    row_ss = _pad_edges(edge_index_s_s[0], e_ss, 0)
    col_ss = _pad_edges(edge_index_s_s[1], e_ss, N_STATE)

    ones128 = jnp.ones((CHUNK,), f32)
    zer_deg = jnp.zeros((64,), f32)

    deg_k = _make_degrees((e_vv, e_ss, e_in), (NG_ACC, NS_ACC, NS_ACC))
    degp_vv, degp_ss, cntp_in = deg_k(col_vv, col_ss, col_in, ones128, zer_deg)
    deg_vv = degp_vv[:NG_ACC] + degp_vv[NG_ACC:]
    deg_ss = degp_ss[:NS_ACC] + degp_ss[NS_ACC:]
    cnt_in = cntp_in[:NS_ACC] + cntp_in[NS_ACC:]
    dinv_g = _safe_rsqrt(deg_vv)[:, None]
    dinv_s = _safe_rsqrt(deg_ss)[:, None]
    inv_cnt = (1.0 / jnp.maximum(cnt_in, 1.0))[:, None]

    x16 = _pad_rows(jnp.pad(game_x, ((0, 0), (0, DG - D_IN))), NG_ACC)
    x_s = x16 * dinv_g
    zer_g = jnp.zeros((64, DG), f32)
    apply_g = _make_apply(1, DG, e_vv, NG_ACC, False)
    p1 = apply_g(x_s, row_vv, col_vv, zer_g)[:, 0]
    h1_s = (dinv_g * dinv_g) * (p1[0] + p1[1])
    p2 = apply_g(h1_s, row_vv, col_vv, zer_g)[:, 0]
    w48 = jnp.concatenate([
        jnp.pad(W1_0, ((0, DG - D_IN), (0, 0))),
        jnp.pad(W1_1, ((0, DG - D_IN), (0, 0))),
        jnp.pad(W1_2, ((0, DG - D_IN), (0, 0)))], axis=0)
    b1 = (b1_0 + b1_1 + b1_2)[None, :]
    g4 = _tc_game(x16, p1, p2, dinv_g, w48, b1)
    gx = [g4[c] for c in range(4)]

    zer_s = jnp.zeros((64, 128), f32)
    apply_h = _make_apply(4, 128, e_hist, NS_ACC, True)
    a3p = apply_h(*gx, row_hist, col_hist, w_hist, zer_s)
    apply_i = _make_apply(4, 128, e_in, NS_ACC, False)
    s4p = apply_i(*gx, row_in, col_in, zer_s)

    x8 = _pad_rows(jnp.pad(state_x, ((0, 0), (0, 8 - D_IN))), NS_ACC)
    w3root8 = jnp.pad(W3_root, ((0, 8 - D_IN), (0, 0)))
    s4, s4s = _tc_s34(a3p, s4p, x8, dinv_s, inv_cnt, W3_rel, w3root8,
                      W4_l, W4_r, b3_rel[None, :], b4_l[None, :])

    apply_s = _make_apply(4, 128, e_ss, NS_ACC, False)
    q1 = apply_s(s4s[0], s4s[1], s4s[2], s4s[3], row_ss, col_ss, zer_s)
    h1, h1s = _tc_hop(q1, dinv_s)
    q2 = apply_s(h1s[0], h1s[1], h1s[2], h1s[3], row_ss, col_ss, zer_s)
    h2, h2s = _tc_hop(q2, dinv_s)
    q3 = apply_s(h2s[0], h2s[1], h2s[2], h2s[3], row_ss, col_ss, zer_s)

    bsum = (b2_0 + b2_1 + b2_2 + b2_3)[None, :]
    out = _tc_final(s4, h1, h2, q3, dinv_s, W2_0, W2_1, W2_2, W2_3,
                    bsum, Wlin, blin[None, :])
    return out[:N_STATE]


# speedup vs baseline: 1.7751x; 1.1647x over previous
import functools

import jax
import jax.numpy as jnp
from jax import lax
from jax.experimental import pallas as pl
from jax.experimental.pallas import tpu as pltpu
from jax.experimental.pallas import tpu_sc as plsc

N_GAME = 50000
N_STATE = 10000
D_IN = 5
H = 512
OUT = 8

NC = 2
NS = 16
NW = NC * NS
CHUNK = 128

NG_ACC = 50176
NS_ACC = 10240
DG = 16


def _pad_rows(x, n):
    return jnp.pad(x, ((0, n - x.shape[0]),) + ((0, 0),) * (x.ndim - 1))


def _make_degrees(e_pads, n_accs):
    mesh = plsc.VectorSubcoreMesh(
        core_axis_name="c", subcore_axis_name="s", num_cores=NC, num_subcores=NS)
    out_type = tuple(jax.ShapeDtypeStruct((NC * n,), jnp.float32) for n in n_accs)
    scratch = [
        pltpu.VMEM((CHUNK,), jnp.int32),
        pltpu.VMEM((CHUNK,), jnp.float32),
        pltpu.VMEM((64,), jnp.float32),
    ] + [pltpu.VMEM_SHARED((n,), jnp.float32) for n in n_accs]

    def body(col0, col1, col2, ones_hbm, zeros_hbm, out0, out1, out2,
             cidx, obuf, sbuf, acc0, acc1, acc2):
        cid = lax.axis_index("c")
        sid = lax.axis_index("s")
        w = cid * NS + sid
        cols = (col0, col1, col2)
        outs = (out0, out1, out2)
        pltpu.sync_copy(ones_hbm, obuf)
        pltpu.sync_copy(zeros_hbm, sbuf)
        acc_refs = (acc0, acc1, acc2)
        if True:
            for i in range(3):
                rows_w = n_accs[i] // NS

                def zbody(t, _, i=i, rows_w=rows_w):
                    pltpu.sync_copy(
                        sbuf, acc_refs[i].at[pl.ds(sid * rows_w + t * 64, 64)])
                    return 0

                lax.fori_loop(0, rows_w // 64, zbody, 0)
            plsc.subcore_barrier()
            for i in range(3):
                ew = e_pads[i] // NW
                nk = ew // CHUNK

                def ebody(k, _, i=i):
                    base = w * ew + k * CHUNK
                    pltpu.sync_copy(cols[i].at[pl.ds(base, CHUNK)], cidx)
                    pltpu.sync_copy(obuf, acc_refs[i].at[cidx], add=True)
                    return 0

                lax.fori_loop(0, nk, ebody, 0)
            plsc.subcore_barrier()
            for i in range(3):
                rows_w = n_accs[i] // NS

                def cbody(t, _, i=i, rows_w=rows_w):
                    pltpu.sync_copy(
                        acc_refs[i].at[pl.ds(sid * rows_w + t * 64, 64)], sbuf)
                    pltpu.sync_copy(
                        sbuf,
                        outs[i].at[pl.ds(
                            cid * n_accs[i] + sid * rows_w + t * 64, 64)])
                    return 0

                lax.fori_loop(0, rows_w // 64, cbody, 0)

    return pl.kernel(body, out_type=out_type, mesh=mesh, scratch_types=scratch)


def _make_apply(n_chunks, dc, e_pad, n_acc, weighted, zstrip):
    mesh = plsc.VectorSubcoreMesh(
        core_axis_name="c", subcore_axis_name="s", num_cores=NC, num_subcores=NS)
    out_type = jax.ShapeDtypeStruct((NC, n_chunks, n_acc, dc), jnp.float32)
    rows_w = n_acc // NS
    nz = rows_w // zstrip
    nsb_tot = e_pad // (8 * CHUNK)
    base, rem = nsb_tot // NW, nsb_tot % NW
    scratch = [
        pltpu.VMEM((8 * CHUNK,), jnp.int32),
        pltpu.VMEM((8, CHUNK), jnp.int32),
        pltpu.VMEM((CHUNK, dc), jnp.float32),
        pltpu.VMEM((zstrip, dc), jnp.float32),
    ]
    if weighted:
        scratch.append(pltpu.VMEM((8 * CHUNK,), jnp.float32))
    scratch.append(pltpu.VMEM_SHARED((n_acc, dc), jnp.float32))
    cparams = pltpu.CompilerParams(use_tc_tiling_on_sc=(dc % 128 == 0),
                                   needs_layout_passes=False)

    def body(*refs):
        xs = refs[:n_chunks]
        row_hbm, col_hbm = refs[n_chunks], refs[n_chunks + 1]
        p = n_chunks + 2
        if weighted:
            wts_hbm = refs[p]
            p += 1
        zeros_hbm = refs[p]
        out = refs[p + 1]
        (ridx, cidx, dbuf, zbuf) = refs[p + 2:p + 6]
        if weighted:
            wbuf = refs[p + 6]
        acc = refs[-1]
        cid = lax.axis_index("c")
        sid = lax.axis_index("s")
        w = cid * NS + sid
        nsb_w = base + jnp.where(w < rem, 1, 0)
        sb0 = w * base + jnp.minimum(w, rem)

        pltpu.sync_copy(zeros_hbm, zbuf)

        for c in range(n_chunks):
            for t in range(nz):
                pltpu.sync_copy(
                    zbuf, acc.at[pl.ds(sid * rows_w + t * zstrip, zstrip)])
            plsc.subcore_barrier()

            def sbody(sbi, _, c=c):
                sb = sb0 + sbi
                pltpu.sync_copy(row_hbm.at[pl.ds(sb * (8 * CHUNK), 8 * CHUNK)],
                                ridx)
                pltpu.sync_copy(
                    col_hbm.at[pl.ds(pl.multiple_of(sb * 8, 8), 8), :], cidx)
                if weighted:
                    pltpu.sync_copy(
                        wts_hbm.at[pl.ds(sb * (8 * CHUNK), 8 * CHUNK)], wbuf)
                for j in range(8):
                    pltpu.sync_copy(xs[c].at[ridx.at[pl.ds(j * CHUNK, CHUNK)]],
                                    dbuf)
                    if weighted:
                        def mbody(e, _, j=j):
                            ws = plsc.load_gather(
                                wbuf,
                                [jnp.full((16,), j * CHUNK + e, jnp.int32)])
                            for q in range(dc // 16):
                                dbuf[e, pl.ds(q * 16, 16)] = (
                                    dbuf[e, pl.ds(q * 16, 16)] * ws)
                            return 0

                        lax.fori_loop(0, CHUNK, mbody, 0)
                    pltpu.sync_copy(dbuf, acc.at[cidx.at[j]], add=True)
                return 0

            lax.fori_loop(0, nsb_w, sbody, 0)
            plsc.subcore_barrier()

            for t in range(nz):
                r0 = sid * rows_w + t * zstrip
                pltpu.sync_copy(acc.at[pl.ds(r0, zstrip)],
                                dbuf.at[pl.ds(0, zstrip)])
                pltpu.sync_copy(dbuf.at[pl.ds(0, zstrip)],
                                out.at[cid, c, pl.ds(r0, zstrip)])
            plsc.subcore_barrier()

    return pl.kernel(body, out_type=out_type, mesh=mesh, scratch_types=scratch,
                     compiler_params=cparams)


_R = 512


def _full(shape):
    return pl.BlockSpec(shape, lambda i: (0,) * len(shape))


def _dot(a, b):
    return jnp.dot(a, b, preferred_element_type=jnp.float32)


def _tg_body(x_ref, p1_ref, p2_ref, dinv_ref, w_ref, b_ref, out_ref):
    dinv = dinv_ref[...]
    h1 = dinv * (p1_ref[0] + p1_ref[1])
    h2 = dinv * (p2_ref[0] + p2_ref[1])
    a = jnp.concatenate([x_ref[...], h1, h2], axis=1)
    g = jnp.maximum(_dot(a, w_ref[...]) + b_ref[...], 0.0)
    for c in range(4):
        out_ref[c] = g[:, c * 128:(c + 1) * 128]


def _tc_game(x16, p1, p2, dinv, w48, bias):
    grid = (NG_ACC // _R,)
    return pl.pallas_call(
        _tg_body,
        grid=grid,
        in_specs=[
            pl.BlockSpec((_R, DG), lambda i: (i, 0)),
            pl.BlockSpec((2, _R, DG), lambda i: (0, i, 0)),
            pl.BlockSpec((2, _R, DG), lambda i: (0, i, 0)),
            pl.BlockSpec((_R, 1), lambda i: (i, 0)),
            _full((3 * DG, H)),
            _full((1, H)),
        ],
        out_specs=pl.BlockSpec((4, _R, 128), lambda i: (0, i, 0)),
        out_shape=jax.ShapeDtypeStruct((4, NG_ACC, 128), jnp.float32),
    )(x16, p1, p2, dinv, w48, bias)


def _combine(p_ref, c):
    return jnp.concatenate([p_ref[0, c2] + p_ref[1, c2] for c2 in range(4)],
                           axis=1) if c is None else p_ref[0, c] + p_ref[1, c]


def _ts_body(a3_ref, s4_ref, x8_ref, dinv_ref, icnt_ref, w3rel_ref, w3root_ref,
             w4l_ref, w4r_ref, b3_ref, b4_ref, s4_out, s4s_out):
    agg = _combine(a3_ref, None)
    s3 = jnp.maximum(
        _dot(agg, w3rel_ref[...]) + _dot(x8_ref[...], w3root_ref[...])
        + b3_ref[...], 0.0)
    mean = _combine(s4_ref, None) * icnt_ref[...]
    s4 = jnp.maximum(
        _dot(mean, w4l_ref[...]) + _dot(s3, w4r_ref[...]) + b4_ref[...], 0.0)
    s4_out[...] = s4
    dinv = dinv_ref[...]
    for c in range(4):
        s4s_out[c] = dinv * s4[:, c * 128:(c + 1) * 128]


def _tc_s34(a3p, s4p, x8, dinv_s, inv_cnt, w3rel, w3root8, w4l, w4r, b3, b4):
    grid = (NS_ACC // _R,)
    return pl.pallas_call(
        _ts_body,
        grid=grid,
        in_specs=[
            pl.BlockSpec((2, 4, _R, 128), lambda i: (0, 0, i, 0)),
            pl.BlockSpec((2, 4, _R, 128), lambda i: (0, 0, i, 0)),
            pl.BlockSpec((_R, 8), lambda i: (i, 0)),
            pl.BlockSpec((_R, 1), lambda i: (i, 0)),
            pl.BlockSpec((_R, 1), lambda i: (i, 0)),
            _full((H, H)),
            _full((8, H)),
            _full((H, H)),
            _full((H, H)),
            _full((1, H)),
            _full((1, H)),
        ],
        out_specs=[
            pl.BlockSpec((_R, H), lambda i: (i, 0)),
            pl.BlockSpec((4, _R, 128), lambda i: (0, i, 0)),
        ],
        out_shape=[
            jax.ShapeDtypeStruct((NS_ACC, H), jnp.float32),
            jax.ShapeDtypeStruct((4, NS_ACC, 128), jnp.float32),
        ],
    )(a3p, s4p, x8, dinv_s, inv_cnt, w3rel, w3root8, w4l, w4r, b3, b4)


def _thc_body(qp_ref, dinv_ref, h_out, hs_out):
    dinv = dinv_ref[...]
    for c in range(4):
        hc = dinv * _combine(qp_ref, c)
        h_out[:, c * 128:(c + 1) * 128] = hc
        hs_out[c] = dinv * hc


def _tc_hop(qp, dinv_s):
    grid = (NS_ACC // _R,)
    return pl.pallas_call(
        _thc_body,
        grid=grid,
        in_specs=[
            pl.BlockSpec((2, 4, _R, 128), lambda i: (0, 0, i, 0)),
            pl.BlockSpec((_R, 1), lambda i: (i, 0)),
        ],
        out_specs=[
            pl.BlockSpec((_R, H), lambda i: (i, 0)),
            pl.BlockSpec((4, _R, 128), lambda i: (0, i, 0)),
        ],
        out_shape=[
            jax.ShapeDtypeStruct((NS_ACC, H), jnp.float32),
            jax.ShapeDtypeStruct((4, NS_ACC, 128), jnp.float32),
        ],
    )(qp, dinv_s)


def _tf_body(s4_ref, h1_ref, h2_ref, q3_ref, dinv_ref, w0_ref, w1_ref, w2_ref,
             w3_ref, b_ref, wlin_ref, blin_ref, out_ref):
    h3 = dinv_ref[...] * _combine(q3_ref, None)
    hh = (_dot(s4_ref[...], w0_ref[...]) + _dot(h1_ref[...], w1_ref[...])
          + _dot(h2_ref[...], w2_ref[...]) + _dot(h3, w3_ref[...])
          + b_ref[...])
    hh = jnp.maximum(hh, 0.0)
    out_ref[...] = _dot(hh, wlin_ref[...]) + blin_ref[...]


def _tc_final(s4, h1, h2, q3p, dinv_s, w0, w1, w2, w3, bsum, wlin, blin):
    grid = (NS_ACC // _R,)
    return pl.pallas_call(
        _tf_body,
        grid=grid,
        in_specs=[
            pl.BlockSpec((_R, H), lambda i: (i, 0)),
            pl.BlockSpec((_R, H), lambda i: (i, 0)),
            pl.BlockSpec((_R, H), lambda i: (i, 0)),
            pl.BlockSpec((2, 4, _R, 128), lambda i: (0, 0, i, 0)),
            pl.BlockSpec((_R, 1), lambda i: (i, 0)),
            _full((H, H)),
            _full((H, H)),
            _full((H, H)),
            _full((H, H)),
            _full((1, H)),
            _full((H, OUT)),
            _full((1, OUT)),
        ],
        out_specs=pl.BlockSpec((_R, OUT), lambda i: (i, 0)),
        out_shape=jax.ShapeDtypeStruct((NS_ACC, OUT), jnp.float32),
    )(s4, h1, h2, q3p, dinv_s, w0, w1, w2, w3, bsum, wlin, blin)


def _pad_edges(idx_arr, e_pad, fill):
    return jnp.pad(idx_arr, (0, e_pad - idx_arr.shape[0]),
                   constant_values=fill)


def _epad(e):
    g = NW * CHUNK
    return ((e + g - 1) // g) * g


def _safe_rsqrt(deg):
    return jnp.where(deg > 0, lax.rsqrt(jnp.maximum(deg, 1.0)), 0.0)


def kernel(game_x, state_x, edge_index_v_v, edge_index_history_v_s,
           edge_attr_history_v_s, edge_index_in_v_s, edge_index_s_s,
           W1_0, b1_0, W1_1, b1_1, W1_2, b1_2, W2_0, b2_0, W2_1, b2_1,
           W2_2, b2_2, W2_3, b2_3, W3_rel, b3_rel, W3_root, W4_l, b4_l,
           W4_r, Wlin, blin):
    f32 = jnp.float32
    e_vv = _epad(edge_index_v_v.shape[1])
    e_hist = _epad(edge_index_history_v_s.shape[1])
    e_in = _epad(edge_index_in_v_s.shape[1])
    e_ss = _epad(edge_index_s_s.shape[1])

    row_vv = _pad_edges(edge_index_v_v[0], e_vv, 0)
    col_vv = _pad_edges(edge_index_v_v[1], e_vv, N_GAME)
    row_hist = _pad_edges(edge_index_history_v_s[0], e_hist, 0)
    col_hist = _pad_edges(edge_index_history_v_s[1], e_hist, N_STATE)
    w_hist = jnp.pad(edge_attr_history_v_s, (0, e_hist - edge_attr_history_v_s.shape[0]))
    row_in = _pad_edges(edge_index_in_v_s[0], e_in, 0)
    col_in = _pad_edges(edge_index_in_v_s[1], e_in, N_STATE)
    row_ss = _pad_edges(edge_index_s_s[0], e_ss, 0)
    col_ss = _pad_edges(edge_index_s_s[1], e_ss, N_STATE)

    ones128 = jnp.ones((CHUNK,), f32)
    zer_deg = jnp.zeros((64,), f32)

    deg_k = _make_degrees((e_vv, e_ss, e_in), (NG_ACC, NS_ACC, NS_ACC))
    degp_vv, degp_ss, cntp_in = deg_k(col_vv, col_ss, col_in, ones128, zer_deg)
    deg_vv = degp_vv[:NG_ACC] + degp_vv[NG_ACC:]
    deg_ss = degp_ss[:NS_ACC] + degp_ss[NS_ACC:]
    cnt_in = cntp_in[:NS_ACC] + cntp_in[NS_ACC:]
    dinv_g = _safe_rsqrt(deg_vv)[:, None]
    dinv_s = _safe_rsqrt(deg_ss)[:, None]
    inv_cnt = (1.0 / jnp.maximum(cnt_in, 1.0))[:, None]

    x16 = _pad_rows(jnp.pad(game_x, ((0, 0), (0, DG - D_IN))), NG_ACC)
    x_s = x16 * dinv_g
    zer_g = jnp.zeros((112, DG), f32)
    apply_g = _make_apply(1, DG, e_vv, NG_ACC, False, 112)
    col_vv2 = col_vv.reshape(-1, CHUNK)
    p1 = apply_g(x_s, row_vv, col_vv2, zer_g)[:, 0]
    h1_s = (dinv_g * dinv_g) * (p1[0] + p1[1])
    p2 = apply_g(h1_s, row_vv, col_vv2, zer_g)[:, 0]
    w48 = jnp.concatenate([
        jnp.pad(W1_0, ((0, DG - D_IN), (0, 0))),
        jnp.pad(W1_1, ((0, DG - D_IN), (0, 0))),
        jnp.pad(W1_2, ((0, DG - D_IN), (0, 0)))], axis=0)
    b1 = (b1_0 + b1_1 + b1_2)[None, :]
    g4 = _tc_game(x16, p1, p2, dinv_g, w48, b1)
    gx = [g4[c] for c in range(4)]

    zer_s = jnp.zeros((128, 128), f32)
    apply_h = _make_apply(4, 128, e_hist, NS_ACC, True, 128)
    a3p = apply_h(*gx, row_hist, col_hist.reshape(-1, CHUNK), w_hist, zer_s)
    apply_i = _make_apply(4, 128, e_in, NS_ACC, False, 128)
    s4p = apply_i(*gx, row_in, col_in.reshape(-1, CHUNK), zer_s)

    x8 = _pad_rows(jnp.pad(state_x, ((0, 0), (0, 8 - D_IN))), NS_ACC)
    w3root8 = jnp.pad(W3_root, ((0, 8 - D_IN), (0, 0)))
    s4, s4s = _tc_s34(a3p, s4p, x8, dinv_s, inv_cnt, W3_rel, w3root8,
                      W4_l, W4_r, b3_rel[None, :], b4_l[None, :])

    apply_s = _make_apply(4, 128, e_ss, NS_ACC, False, 128)
    col_ss2 = col_ss.reshape(-1, CHUNK)
    q1 = apply_s(s4s[0], s4s[1], s4s[2], s4s[3], row_ss, col_ss2, zer_s)
    h1, h1s = _tc_hop(q1, dinv_s)
    q2 = apply_s(h1s[0], h1s[1], h1s[2], h1s[3], row_ss, col_ss2, zer_s)
    h2, h2s = _tc_hop(q2, dinv_s)
    q3 = apply_s(h2s[0], h2s[1], h2s[2], h2s[3], row_ss, col_ss2, zer_s)

    bsum = (b2_0 + b2_1 + b2_2 + b2_3)[None, :]
    out = _tc_final(s4, h1, h2, q3, dinv_s, W2_0, W2_1, W2_2, W2_3,
                    bsum, Wlin, blin[None, :])
    return out[:N_STATE]


# speedup vs baseline: 1.8797x; 1.0589x over previous
import functools

import jax
import jax.numpy as jnp
from jax import lax
from jax.experimental import pallas as pl
from jax.experimental.pallas import tpu as pltpu
from jax.experimental.pallas import tpu_sc as plsc

N_GAME = 50000
N_STATE = 10000
D_IN = 5
H = 512
OUT = 8

NC = 2
NS = 16
NW = NC * NS
CHUNK = 128

NG_ACC = 50176
NS_ACC = 10240
DG = 16


def _pad_rows(x, n):
    return jnp.pad(x, ((0, n - x.shape[0]),) + ((0, 0),) * (x.ndim - 1))


def _make_degrees(e_pads, n_accs):
    mesh = plsc.VectorSubcoreMesh(
        core_axis_name="c", subcore_axis_name="s", num_cores=NC, num_subcores=NS)
    out_type = tuple(jax.ShapeDtypeStruct((NC * n,), jnp.float32) for n in n_accs)
    scratch = [
        pltpu.VMEM((8, CHUNK), jnp.int32),
        pltpu.VMEM((CHUNK,), jnp.float32),
        pltpu.VMEM((64,), jnp.float32),
    ] + [pltpu.VMEM_SHARED((n,), jnp.float32) for n in n_accs]
    dists = []
    for e in e_pads:
        nsb_tot = e // (8 * CHUNK)
        dists.append((nsb_tot // NW, nsb_tot % NW))

    def body(col0, col1, col2, ones_hbm, zeros_hbm, out0, out1, out2,
             cidx, obuf, sbuf, acc0, acc1, acc2):
        cid = lax.axis_index("c")
        sid = lax.axis_index("s")
        w = cid * NS + sid
        cols = (col0, col1, col2)
        outs = (out0, out1, out2)
        pltpu.sync_copy(ones_hbm, obuf)
        pltpu.sync_copy(zeros_hbm, sbuf)
        acc_refs = (acc0, acc1, acc2)
        if True:
            for i in range(3):
                rows_w = n_accs[i] // NS

                def zbody(t, _, i=i, rows_w=rows_w):
                    pltpu.sync_copy(
                        sbuf, acc_refs[i].at[pl.ds(sid * rows_w + t * 64, 64)])
                    return 0

                lax.fori_loop(0, rows_w // 64, zbody, 0)
            plsc.subcore_barrier()
            for i in range(3):
                base_i, rem_i = dists[i]
                nsb_w = base_i + jnp.where(w < rem_i, 1, 0)
                sb0 = w * base_i + jnp.minimum(w, rem_i)

                def sbody(sbi, _, i=i, sb0=sb0):
                    sb = sb0 + sbi
                    pltpu.sync_copy(
                        cols[i].at[pl.ds(pl.multiple_of(sb * 8, 8), 8), :],
                        cidx)
                    for j in range(8):
                        pltpu.sync_copy(obuf, acc_refs[i].at[cidx.at[j]],
                                        add=True)
                    return 0

                lax.fori_loop(0, nsb_w, sbody, 0)
            plsc.subcore_barrier()
            for i in range(3):
                rows_w = n_accs[i] // NS

                def cbody(t, _, i=i, rows_w=rows_w):
                    pltpu.sync_copy(
                        acc_refs[i].at[pl.ds(sid * rows_w + t * 64, 64)], sbuf)
                    pltpu.sync_copy(
                        sbuf,
                        outs[i].at[pl.ds(
                            cid * n_accs[i] + sid * rows_w + t * 64, 64)])
                    return 0

                lax.fori_loop(0, rows_w // 64, cbody, 0)

    return pl.kernel(body, out_type=out_type, mesh=mesh, scratch_types=scratch)


def _make_apply(n_chunks, dc, jobs, n_acc, zstrip):
    mesh = plsc.VectorSubcoreMesh(
        core_axis_name="c", subcore_axis_name="s", num_cores=NC, num_subcores=NS)
    out_type = tuple(jax.ShapeDtypeStruct((NC, n_chunks, n_acc, dc),
                                          jnp.float32) for _ in jobs)
    rows_w = n_acc // NS
    nz = rows_w // zstrip
    dists = []
    for e_pad, _ in jobs:
        nsb_tot = e_pad // (8 * CHUNK)
        dists.append((nsb_tot // NW, nsb_tot % NW))
    any_wtd = any(wtd for _, wtd in jobs)
    scratch = [
        pltpu.VMEM((8 * CHUNK,), jnp.int32),
        pltpu.VMEM((8, CHUNK), jnp.int32),
        pltpu.VMEM((CHUNK, dc), jnp.float32),
        pltpu.VMEM((zstrip, dc), jnp.float32),
    ]
    if any_wtd:
        scratch.append(pltpu.VMEM((8 * CHUNK,), jnp.float32))
    scratch.append(pltpu.VMEM_SHARED((n_acc, dc), jnp.float32))
    cparams = pltpu.CompilerParams(use_tc_tiling_on_sc=(dc % 128 == 0),
                                   needs_layout_passes=False)

    def body(*refs):
        xs = refs[:n_chunks]
        p = n_chunks
        job_refs = []
        for _, wtd in jobs:
            row_hbm, col_hbm = refs[p], refs[p + 1]
            p += 2
            wts_hbm = None
            if wtd:
                wts_hbm = refs[p]
                p += 1
            job_refs.append((row_hbm, col_hbm, wts_hbm))
        zeros_hbm = refs[p]
        p += 1
        outs = refs[p:p + len(jobs)]
        p += len(jobs)
        (ridx, cidx, dbuf, zbuf) = refs[p:p + 4]
        if any_wtd:
            wbuf = refs[p + 4]
        acc = refs[-1]
        cid = lax.axis_index("c")
        sid = lax.axis_index("s")
        w = cid * NS + sid

        pltpu.sync_copy(zeros_hbm, zbuf)

        for c in range(n_chunks):
            for ji, (row_hbm, col_hbm, wts_hbm) in enumerate(job_refs):
                base_j, rem_j = dists[ji]
                nsb_w = base_j + jnp.where(w < rem_j, 1, 0)
                sb0 = w * base_j + jnp.minimum(w, rem_j)
                for t in range(nz):
                    pltpu.sync_copy(
                        zbuf, acc.at[pl.ds(sid * rows_w + t * zstrip, zstrip)])
                plsc.subcore_barrier()

                def sbody(sbi, _, c=c, row_hbm=row_hbm, col_hbm=col_hbm,
                          wts_hbm=wts_hbm, sb0=sb0):
                    sb = sb0 + sbi
                    pltpu.sync_copy(
                        row_hbm.at[pl.ds(sb * (8 * CHUNK), 8 * CHUNK)], ridx)
                    pltpu.sync_copy(
                        col_hbm.at[pl.ds(pl.multiple_of(sb * 8, 8), 8), :],
                        cidx)
                    if wts_hbm is not None:
                        pltpu.sync_copy(
                            wts_hbm.at[pl.ds(sb * (8 * CHUNK), 8 * CHUNK)],
                            wbuf)
                    for j in range(8):
                        pltpu.sync_copy(
                            xs[c].at[ridx.at[pl.ds(j * CHUNK, CHUNK)]], dbuf)
                        if wts_hbm is not None:
                            def mbody(e, _, j=j):
                                ws = plsc.load_gather(
                                    wbuf,
                                    [jnp.full((16,), j * CHUNK + e, jnp.int32)])
                                for q in range(dc // 16):
                                    dbuf[e, pl.ds(q * 16, 16)] = (
                                        dbuf[e, pl.ds(q * 16, 16)] * ws)
                                return 0

                            lax.fori_loop(0, CHUNK, mbody, 0)
                        pltpu.sync_copy(dbuf, acc.at[cidx.at[j]], add=True)
                    return 0

                lax.fori_loop(0, nsb_w, sbody, 0)
                plsc.subcore_barrier()

                for t in range(nz):
                    r0 = sid * rows_w + t * zstrip
                    pltpu.sync_copy(acc.at[pl.ds(r0, zstrip)],
                                    dbuf.at[pl.ds(0, zstrip)])
                    pltpu.sync_copy(dbuf.at[pl.ds(0, zstrip)],
                                    outs[ji].at[cid, c, pl.ds(r0, zstrip)])
                plsc.subcore_barrier()

    return pl.kernel(body, out_type=out_type, mesh=mesh, scratch_types=scratch,
                     compiler_params=cparams)


_R = 512


def _full(shape):
    return pl.BlockSpec(shape, lambda i: (0,) * len(shape))


def _dot(a, b):
    return jnp.dot(a, b, preferred_element_type=jnp.float32)


def _tg_body(x_ref, p1_ref, p2_ref, dinv_ref, w_ref, b_ref, out_ref):
    dinv = dinv_ref[...]
    h1 = dinv * (p1_ref[0] + p1_ref[1])
    h2 = dinv * (p2_ref[0] + p2_ref[1])
    a = jnp.concatenate([x_ref[...], h1, h2], axis=1)
    g = jnp.maximum(_dot(a, w_ref[...]) + b_ref[...], 0.0)
    for c in range(4):
        out_ref[c] = g[:, c * 128:(c + 1) * 128]


def _tc_game(x16, p1, p2, dinv, w48, bias):
    grid = (NG_ACC // _R,)
    return pl.pallas_call(
        _tg_body,
        grid=grid,
        in_specs=[
            pl.BlockSpec((_R, DG), lambda i: (i, 0)),
            pl.BlockSpec((2, _R, DG), lambda i: (0, i, 0)),
            pl.BlockSpec((2, _R, DG), lambda i: (0, i, 0)),
            pl.BlockSpec((_R, 1), lambda i: (i, 0)),
            _full((3 * DG, H)),
            _full((1, H)),
        ],
        out_specs=pl.BlockSpec((4, _R, 128), lambda i: (0, i, 0)),
        out_shape=jax.ShapeDtypeStruct((4, NG_ACC, 128), jnp.float32),
    )(x16, p1, p2, dinv, w48, bias)


def _combine(p_ref, c):
    return jnp.concatenate([p_ref[0, c2] + p_ref[1, c2] for c2 in range(4)],
                           axis=1) if c is None else p_ref[0, c] + p_ref[1, c]


def _ts_body(a3_ref, s4_ref, x8_ref, dinv_ref, icnt_ref, w3rel_ref, w3root_ref,
             w4l_ref, w4r_ref, b3_ref, b4_ref, s4_out, s4s_out):
    agg = _combine(a3_ref, None)
    s3 = jnp.maximum(
        _dot(agg, w3rel_ref[...]) + _dot(x8_ref[...], w3root_ref[...])
        + b3_ref[...], 0.0)
    mean = _combine(s4_ref, None) * icnt_ref[...]
    s4 = jnp.maximum(
        _dot(mean, w4l_ref[...]) + _dot(s3, w4r_ref[...]) + b4_ref[...], 0.0)
    s4_out[...] = s4
    dinv = dinv_ref[...]
    for c in range(4):
        s4s_out[c] = dinv * s4[:, c * 128:(c + 1) * 128]


def _tc_s34(a3p, s4p, x8, dinv_s, inv_cnt, w3rel, w3root8, w4l, w4r, b3, b4):
    grid = (NS_ACC // _R,)
    return pl.pallas_call(
        _ts_body,
        grid=grid,
        in_specs=[
            pl.BlockSpec((2, 4, _R, 128), lambda i: (0, 0, i, 0)),
            pl.BlockSpec((2, 4, _R, 128), lambda i: (0, 0, i, 0)),
            pl.BlockSpec((_R, 8), lambda i: (i, 0)),
            pl.BlockSpec((_R, 1), lambda i: (i, 0)),
            pl.BlockSpec((_R, 1), lambda i: (i, 0)),
            _full((H, H)),
            _full((8, H)),
            _full((H, H)),
            _full((H, H)),
            _full((1, H)),
            _full((1, H)),
        ],
        out_specs=[
            pl.BlockSpec((_R, H), lambda i: (i, 0)),
            pl.BlockSpec((4, _R, 128), lambda i: (0, i, 0)),
        ],
        out_shape=[
            jax.ShapeDtypeStruct((NS_ACC, H), jnp.float32),
            jax.ShapeDtypeStruct((4, NS_ACC, 128), jnp.float32),
        ],
    )(a3p, s4p, x8, dinv_s, inv_cnt, w3rel, w3root8, w4l, w4r, b3, b4)


def _thc_body(qp_ref, dinv_ref, h_out, hs_out):
    dinv = dinv_ref[...]
    for c in range(4):
        hc = dinv * _combine(qp_ref, c)
        h_out[:, c * 128:(c + 1) * 128] = hc
        hs_out[c] = dinv * hc


def _tc_hop(qp, dinv_s):
    grid = (NS_ACC // _R,)
    return pl.pallas_call(
        _thc_body,
        grid=grid,
        in_specs=[
            pl.BlockSpec((2, 4, _R, 128), lambda i: (0, 0, i, 0)),
            pl.BlockSpec((_R, 1), lambda i: (i, 0)),
        ],
        out_specs=[
            pl.BlockSpec((_R, H), lambda i: (i, 0)),
            pl.BlockSpec((4, _R, 128), lambda i: (0, i, 0)),
        ],
        out_shape=[
            jax.ShapeDtypeStruct((NS_ACC, H), jnp.float32),
            jax.ShapeDtypeStruct((4, NS_ACC, 128), jnp.float32),
        ],
    )(qp, dinv_s)


def _tf_body(s4_ref, h1_ref, h2_ref, q3_ref, dinv_ref, w0_ref, w1_ref, w2_ref,
             w3_ref, b_ref, wlin_ref, blin_ref, out_ref):
    h3 = dinv_ref[...] * _combine(q3_ref, None)
    hh = (_dot(s4_ref[...], w0_ref[...]) + _dot(h1_ref[...], w1_ref[...])
          + _dot(h2_ref[...], w2_ref[...]) + _dot(h3, w3_ref[...])
          + b_ref[...])
    hh = jnp.maximum(hh, 0.0)
    out_ref[...] = _dot(hh, wlin_ref[...]) + blin_ref[...]


def _tc_final(s4, h1, h2, q3p, dinv_s, w0, w1, w2, w3, bsum, wlin, blin):
    grid = (NS_ACC // _R,)
    return pl.pallas_call(
        _tf_body,
        grid=grid,
        in_specs=[
            pl.BlockSpec((_R, H), lambda i: (i, 0)),
            pl.BlockSpec((_R, H), lambda i: (i, 0)),
            pl.BlockSpec((_R, H), lambda i: (i, 0)),
            pl.BlockSpec((2, 4, _R, 128), lambda i: (0, 0, i, 0)),
            pl.BlockSpec((_R, 1), lambda i: (i, 0)),
            _full((H, H)),
            _full((H, H)),
            _full((H, H)),
            _full((H, H)),
            _full((1, H)),
            _full((H, OUT)),
            _full((1, OUT)),
        ],
        out_specs=pl.BlockSpec((_R, OUT), lambda i: (i, 0)),
        out_shape=jax.ShapeDtypeStruct((NS_ACC, OUT), jnp.float32),
    )(s4, h1, h2, q3p, dinv_s, w0, w1, w2, w3, bsum, wlin, blin)


def _pad_edges(idx_arr, e_pad, fill):
    return jnp.pad(idx_arr, (0, e_pad - idx_arr.shape[0]),
                   constant_values=fill)


def _epad(e):
    g = NW * CHUNK
    return ((e + g - 1) // g) * g


def _safe_rsqrt(deg):
    return jnp.where(deg > 0, lax.rsqrt(jnp.maximum(deg, 1.0)), 0.0)


def kernel(game_x, state_x, edge_index_v_v, edge_index_history_v_s,
           edge_attr_history_v_s, edge_index_in_v_s, edge_index_s_s,
           W1_0, b1_0, W1_1, b1_1, W1_2, b1_2, W2_0, b2_0, W2_1, b2_1,
           W2_2, b2_2, W2_3, b2_3, W3_rel, b3_rel, W3_root, W4_l, b4_l,
           W4_r, Wlin, blin):
    f32 = jnp.float32
    e_vv = _epad(edge_index_v_v.shape[1])
    e_hist = _epad(edge_index_history_v_s.shape[1])
    e_in = _epad(edge_index_in_v_s.shape[1])
    e_ss = _epad(edge_index_s_s.shape[1])

    row_vv = _pad_edges(edge_index_v_v[0], e_vv, 0)
    col_vv = _pad_edges(edge_index_v_v[1], e_vv, N_GAME)
    row_hist = _pad_edges(edge_index_history_v_s[0], e_hist, 0)
    col_hist = _pad_edges(edge_index_history_v_s[1], e_hist, N_STATE)
    w_hist = jnp.pad(edge_attr_history_v_s, (0, e_hist - edge_attr_history_v_s.shape[0]))
    row_in = _pad_edges(edge_index_in_v_s[0], e_in, 0)
    col_in = _pad_edges(edge_index_in_v_s[1], e_in, N_STATE)
    row_ss = _pad_edges(edge_index_s_s[0], e_ss, 0)
    col_ss = _pad_edges(edge_index_s_s[1], e_ss, N_STATE)

    ones128 = jnp.ones((CHUNK,), f32)
    zer_deg = jnp.zeros((64,), f32)

    deg_k = _make_degrees((e_vv, e_ss, e_in), (NG_ACC, NS_ACC, NS_ACC))
    degp_vv, degp_ss, cntp_in = deg_k(
        col_vv.reshape(-1, CHUNK), col_ss.reshape(-1, CHUNK),
        col_in.reshape(-1, CHUNK), ones128, zer_deg)
    deg_vv = degp_vv[:NG_ACC] + degp_vv[NG_ACC:]
    deg_ss = degp_ss[:NS_ACC] + degp_ss[NS_ACC:]
    cnt_in = cntp_in[:NS_ACC] + cntp_in[NS_ACC:]
    dinv_g = _safe_rsqrt(deg_vv)[:, None]
    dinv_s = _safe_rsqrt(deg_ss)[:, None]
    inv_cnt = (1.0 / jnp.maximum(cnt_in, 1.0))[:, None]

    x16 = _pad_rows(jnp.pad(game_x, ((0, 0), (0, DG - D_IN))), NG_ACC)
    x_s = x16 * dinv_g
    zer_g = jnp.zeros((112, DG), f32)
    apply_g = _make_apply(1, DG, ((e_vv, False),), NG_ACC, 112)
    col_vv2 = col_vv.reshape(-1, CHUNK)
    p1 = apply_g(x_s, row_vv, col_vv2, zer_g)[0][:, 0]
    h1_s = (dinv_g * dinv_g) * (p1[0] + p1[1])
    p2 = apply_g(h1_s, row_vv, col_vv2, zer_g)[0][:, 0]
    w48 = jnp.concatenate([
        jnp.pad(W1_0, ((0, DG - D_IN), (0, 0))),
        jnp.pad(W1_1, ((0, DG - D_IN), (0, 0))),
        jnp.pad(W1_2, ((0, DG - D_IN), (0, 0)))], axis=0)
    b1 = (b1_0 + b1_1 + b1_2)[None, :]
    g4 = _tc_game(x16, p1, p2, dinv_g, w48, b1)
    gx = [g4[c] for c in range(4)]

    zer_s = jnp.zeros((128, 128), f32)
    apply_hi = _make_apply(4, 128, ((e_hist, True), (e_in, False)), NS_ACC, 128)
    a3p, s4p = apply_hi(*gx, row_hist, col_hist.reshape(-1, CHUNK), w_hist,
                        row_in, col_in.reshape(-1, CHUNK), zer_s)

    x8 = _pad_rows(jnp.pad(state_x, ((0, 0), (0, 8 - D_IN))), NS_ACC)
    w3root8 = jnp.pad(W3_root, ((0, 8 - D_IN), (0, 0)))
    s4, s4s = _tc_s34(a3p, s4p, x8, dinv_s, inv_cnt, W3_rel, w3root8,
                      W4_l, W4_r, b3_rel[None, :], b4_l[None, :])

    apply_s = _make_apply(4, 128, ((e_ss, False),), NS_ACC, 128)
    col_ss2 = col_ss.reshape(-1, CHUNK)
    q1 = apply_s(s4s[0], s4s[1], s4s[2], s4s[3], row_ss, col_ss2, zer_s)[0]
    h1, h1s = _tc_hop(q1, dinv_s)
    q2 = apply_s(h1s[0], h1s[1], h1s[2], h1s[3], row_ss, col_ss2, zer_s)[0]
    h2, h2s = _tc_hop(q2, dinv_s)
    q3 = apply_s(h2s[0], h2s[1], h2s[2], h2s[3], row_ss, col_ss2, zer_s)[0]

    bsum = (b2_0 + b2_1 + b2_2 + b2_3)[None, :]
    out = _tc_final(s4, h1, h2, q3, dinv_s, W2_0, W2_1, W2_2, W2_3,
                    bsum, Wlin, blin[None, :])
    return out[:N_STATE]
